# baseline (speedup 1.0000x reference)
import functools
import math

import jax
import jax.numpy as jnp
from jax import lax
from jax.experimental import pallas as pl
from jax.experimental.pallas import tpu as pltpu

D = 2048
H = 1408
NE = 8
TOPK = 2
NSH = 2
N = 2048
CAP = 640

_NEG = float("-inf")


def _router_body(x_ref, gw_ref, eff_ref):
    x = x_ref[...]
    gw = gw_ref[...]
    logits = lax.dot_general(x, gw, (((1,), (1,)), ((), ())),
                             preferred_element_type=jnp.float32)

    m = jnp.max(logits, axis=1, keepdims=True)
    ex = jnp.exp(logits - m)
    p = ex / jnp.sum(ex, axis=1, keepdims=True)

    cols = lax.broadcasted_iota(jnp.int32, (N, NE), 1)
    v1 = jnp.max(p, axis=1, keepdims=True)
    i1 = jnp.min(jnp.where(p == v1, cols, NE), axis=1, keepdims=True)
    p2 = jnp.where(cols == i1, _NEG, p)
    v2 = jnp.max(p2, axis=1, keepdims=True)
    i2 = jnp.min(jnp.where(p2 == v2, cols, NE), axis=1, keepdims=True)
    sel = (cols == i1) | (cols == i2)

    W = jnp.where(sel, logits, _NEG)
    Wt = W.T

    jidx = lax.broadcasted_iota(jnp.int32, (1, N), 1)
    CH = 256
    rows = []
    for c in range(N // CH):
        Wc = W[c * CH:(c + 1) * CH, :]
        iidx = c * CH + lax.broadcasted_iota(jnp.int32, (CH, 1), 0)
        ranks = []
        for e in range(NE):
            wi = Wc[:, e:e + 1]
            wj = Wt[e:e + 1, :]
            gt = (wj > wi).astype(jnp.float32)
            eq = ((wj == wi) & (jidx < iidx)).astype(jnp.float32)
            ranks.append(jnp.sum(gt + eq, axis=1, keepdims=True))
        rank = jnp.concatenate(ranks, axis=1)
        keep = sel[c * CH:(c + 1) * CH, :] & (rank < float(CAP))
        eff = jnp.where(keep, logits[c * CH:(c + 1) * CH, :],
                        jnp.float32(0.0))
        rows.append(eff)
    eff_all = jnp.concatenate(rows, axis=0)
    eff_ref[...] = eff_all.T.reshape(NE, 1, N)


def _router(flat, gate_w):
    return pl.pallas_call(
        _router_body,
        out_shape=jax.ShapeDtypeStruct((NE, 1, N), jnp.float32),
    )(flat, gate_w)


MB = 512
HB = 128


def _ffn_body(x_ref, w1_ref, w3_ref, w2_ref, eff_ref, out_ref):
    e = pl.program_id(1)
    h = pl.program_id(2)
    a = x_ref[...]
    w1 = w1_ref[0]
    w3 = w3_ref[0]
    w2 = w2_ref[0]
    g = lax.dot_general(a, w1, (((1,), (1,)), ((), ())),
                        preferred_element_type=jnp.float32)
    u = lax.dot_general(a, w3, (((1,), (1,)), ((), ())),
                        preferred_element_type=jnp.float32)
    hh = (g * jax.nn.sigmoid(g)) * u
    wcol = eff_ref[0, 0, :]
    hh = hh * wcol[:, None]
    part = lax.dot_general(hh, w2, (((1,), (1,)), ((), ())),
                           preferred_element_type=jnp.float32)

    @pl.when((e == 0) & (h == 0))
    def _():
        out_ref[...] = jnp.zeros_like(out_ref)

    out_ref[...] += part


def _expert_ffn(flat, ew1, ew2, ew3, eff):
    grid = (N // MB, NE, H // HB)
    return pl.pallas_call(
        _ffn_body,
        grid=grid,
        in_specs=[
            pl.BlockSpec((MB, D), lambda m, e, h: (m, 0)),
            pl.BlockSpec((1, HB, D), lambda m, e, h: (e, h, 0)),
            pl.BlockSpec((1, HB, D), lambda m, e, h: (e, h, 0)),
            pl.BlockSpec((1, D, HB), lambda m, e, h: (e, 0, h)),
            pl.BlockSpec((1, 1, MB), lambda m, e, h: (e, 0, m)),
        ],
        out_specs=pl.BlockSpec((MB, D), lambda m, e, h: (m, 0)),
        out_shape=jax.ShapeDtypeStruct((N, D), jnp.float32),
        compiler_params=pltpu.CompilerParams(
            dimension_semantics=("arbitrary", "arbitrary", "arbitrary"),
        ),
    )(flat, ew1, ew3, ew2, eff)


def _shared_body(x_ref, w1_ref, w3_ref, w2_ref, prev_ref, out_ref):
    s = pl.program_id(1)
    h = pl.program_id(2)
    a = x_ref[...]
    w1 = w1_ref[0]
    w3 = w3_ref[0]
    w2 = w2_ref[0]
    g = lax.dot_general(a, w1, (((1,), (1,)), ((), ())),
                        preferred_element_type=jnp.float32)
    u = lax.dot_general(a, w3, (((1,), (1,)), ((), ())),
                        preferred_element_type=jnp.float32)
    hh = (g * jax.nn.sigmoid(g)) * u * jnp.float32(1.0 / NSH)
    part = lax.dot_general(hh, w2, (((1,), (1,)), ((), ())),
                           preferred_element_type=jnp.float32)

    @pl.when((s == 0) & (h == 0))
    def _():
        out_ref[...] = prev_ref[...]

    out_ref[...] += part


def _shared_ffn(flat, sw1, sw2, sw3, prev):
    grid = (N // MB, NSH, H // HB)
    return pl.pallas_call(
        _shared_body,
        grid=grid,
        in_specs=[
            pl.BlockSpec((MB, D), lambda m, s, h: (m, 0)),
            pl.BlockSpec((1, HB, D), lambda m, s, h: (s, h, 0)),
            pl.BlockSpec((1, HB, D), lambda m, s, h: (s, h, 0)),
            pl.BlockSpec((1, D, HB), lambda m, s, h: (s, 0, h)),
            pl.BlockSpec((MB, D), lambda m, s, h: (m, 0)),
        ],
        out_specs=pl.BlockSpec((MB, D), lambda m, s, h: (m, 0)),
        out_shape=jax.ShapeDtypeStruct((N, D), jnp.float32),
        compiler_params=pltpu.CompilerParams(
            dimension_semantics=("arbitrary", "arbitrary", "arbitrary"),
        ),
    )(flat, sw1, sw3, sw2, prev)


@jax.jit
def kernel(x, gate_w, gate_b, ew1, ew2, ew3, sw1, sw2, sw3):
    Bb, Tt, Dd = x.shape
    flat = x.reshape(-1, Dd)
    eff = _router(flat, gate_w)
    out_e = _expert_ffn(flat, ew1, ew2, ew3, eff)
    out = _shared_ffn(flat, sw1, sw2, sw3, out_e)
    return out.reshape(Bb, Tt, Dd)


# speedup vs baseline: 1.4826x; 1.4826x over previous
import functools
import math

import jax
import jax.numpy as jnp
from jax import lax
from jax.experimental import pallas as pl
from jax.experimental.pallas import tpu as pltpu
from jax.experimental.pallas import tpu_sc as plsc

D = 2048
H = 1408
NE = 8
TOPK = 2
NSH = 2
N = 2048
CAP = 640
SLOTS = 672
PAD_SLOT = 671
TOT = NE * SLOTS

_NEG = float("-inf")


def _router_body(x_ref, gw_ref, d0_ref, d1_ref, idxl_ref, wslot_ref):
    x = x_ref[...]
    gw = gw_ref[...]
    logits = lax.dot_general(x, gw, (((1,), (1,)), ((), ())),
                             preferred_element_type=jnp.float32)

    m = jnp.max(logits, axis=1, keepdims=True)
    ex = jnp.exp(logits - m)
    p = ex / jnp.sum(ex, axis=1, keepdims=True)

    cols = lax.broadcasted_iota(jnp.int32, (N, NE), 1)
    v1 = jnp.max(p, axis=1, keepdims=True)
    i1 = jnp.min(jnp.where(p == v1, cols, NE), axis=1, keepdims=True)
    p2 = jnp.where(cols == i1, _NEG, p)
    v2 = jnp.max(p2, axis=1, keepdims=True)
    i2 = jnp.min(jnp.where(p2 == v2, cols, NE), axis=1, keepdims=True)
    sel = (cols == i1) | (cols == i2)

    W = jnp.where(sel, logits, _NEG)
    Wt = W.T

    jidx = lax.broadcasted_iota(jnp.int32, (1, N), 1)
    CH = 256
    rank_rows = []
    for c in range(N // CH):
        Wc = W[c * CH:(c + 1) * CH, :]
        iidx = c * CH + lax.broadcasted_iota(jnp.int32, (CH, 1), 0)
        ranks = []
        for e in range(NE):
            wi = Wc[:, e:e + 1]
            wj = Wt[e:e + 1, :]
            gt = (wj > wi).astype(jnp.float32)
            eq = ((wj == wi) & (jidx < iidx)).astype(jnp.float32)
            ranks.append(jnp.sum(gt + eq, axis=1, keepdims=True))
        rank_rows.append(jnp.concatenate(ranks, axis=1))
    rank = jnp.concatenate(rank_rows, axis=0)
    keep = sel & (rank < float(CAP))
    ranki = rank.astype(jnp.int32)

    def dest(ie):
        one = (cols == ie)
        re = jnp.sum(jnp.where(one, ranki, 0), axis=1)
        ke = jnp.any(one & keep, axis=1)
        return ie[:, 0] * SLOTS + jnp.where(ke, re, PAD_SLOT)

    d0_ref[...] = dest(i1)
    d1_ref[...] = dest(i2)

    rows = lax.broadcasted_iota(jnp.int32, (N, SLOTS), 0).astype(jnp.float32)
    slots = lax.broadcasted_iota(jnp.int32, (N, SLOTS), 1).astype(jnp.float32)
    for e in range(NE):
        onehot = (keep[:, e:e + 1]
                  & (rank[:, e:e + 1] == slots)).astype(jnp.float32)
        idx_e = jnp.sum(rows * onehot, axis=0)
        w_e = jnp.sum(logits[:, e:e + 1] * onehot, axis=0)
        idxl_ref[e, 0, :] = idx_e.astype(jnp.int32)
        wslot_ref[e, 0, :] = w_e


def _router(flat, gate_w):
    return pl.pallas_call(
        _router_body,
        out_shape=(
            jax.ShapeDtypeStruct((N,), jnp.int32),
            jax.ShapeDtypeStruct((N,), jnp.int32),
            jax.ShapeDtypeStruct((NE, 1, SLOTS), jnp.int32),
            jax.ShapeDtypeStruct((NE, 1, SLOTS), jnp.float32),
        ),
    )(flat, gate_w)


NW = 32
DCH = 24
DPW = TOT // NW


def _dispatch_body(flat_hbm, idxl_hbm, out_hbm, idx_v, rows_v, sem):
    wid = lax.axis_index("s") * 2 + lax.axis_index("c")
    base = wid * DPW

    def body(c, carry):
        off = base + c * DCH
        pltpu.sync_copy(idxl_hbm.at[pl.ds(off, DCH)], idx_v)
        pltpu.async_copy(flat_hbm.at[idx_v], rows_v, sem).wait()
        pltpu.sync_copy(rows_v, out_hbm.at[pl.ds(off, DCH)])
        return carry

    lax.fori_loop(0, DPW // DCH, body, 0)


def _dispatch(flat, idxl):
    f = functools.partial(
        pl.kernel,
        out_type=jax.ShapeDtypeStruct((TOT, D), jnp.float32),
        mesh=plsc.VectorSubcoreMesh(core_axis_name="c", subcore_axis_name="s"),
        scratch_types=[
            pltpu.VMEM((DCH,), jnp.int32),
            pltpu.VMEM((DCH, D), jnp.float32),
            pltpu.SemaphoreType.DMA,
        ],
    )(_dispatch_body)
    return f(flat, idxl)


HB = 128


def _ffn_body(xg_ref, w1_ref, w3_ref, w2_ref, wslot_ref, out_ref):
    h = pl.program_id(1)
    a = xg_ref[...]
    w1 = w1_ref[0]
    w3 = w3_ref[0]
    w2 = w2_ref[0]
    g = lax.dot_general(a, w1, (((1,), (1,)), ((), ())),
                        preferred_element_type=jnp.float32)
    u = lax.dot_general(a, w3, (((1,), (1,)), ((), ())),
                        preferred_element_type=jnp.float32)
    hh = (g * jax.nn.sigmoid(g)) * u
    hh = hh * wslot_ref[0, 0, :][:, None]
    part = lax.dot_general(hh, w2, (((1,), (1,)), ((), ())),
                           preferred_element_type=jnp.float32)

    @pl.when(h == 0)
    def _():
        out_ref[...] = jnp.zeros_like(out_ref)

    out_ref[...] += part


def _expert_ffn(xg, ew1, ew2, ew3, wslot):
    grid = (NE, H // HB)
    return pl.pallas_call(
        _ffn_body,
        grid=grid,
        in_specs=[
            pl.BlockSpec((SLOTS, D), lambda e, h: (e, 0)),
            pl.BlockSpec((1, HB, D), lambda e, h: (e, h, 0)),
            pl.BlockSpec((1, HB, D), lambda e, h: (e, h, 0)),
            pl.BlockSpec((1, D, HB), lambda e, h: (e, 0, h)),
            pl.BlockSpec((1, 1, SLOTS), lambda e, h: (e, 0, 0)),
        ],
        out_specs=pl.BlockSpec((SLOTS, D), lambda e, h: (e, 0)),
        out_shape=jax.ShapeDtypeStruct((TOT, D), jnp.float32),
        compiler_params=pltpu.CompilerParams(
            dimension_semantics=("arbitrary", "arbitrary"),
        ),
    )(xg, ew1, ew3, ew2, wslot)


MB = 512


def _shared_body(x_ref, w1_ref, w3_ref, w2_ref, out_ref):
    s = pl.program_id(1)
    h = pl.program_id(2)
    a = x_ref[...]
    w1 = w1_ref[0]
    w3 = w3_ref[0]
    w2 = w2_ref[0]
    g = lax.dot_general(a, w1, (((1,), (1,)), ((), ())),
                        preferred_element_type=jnp.float32)
    u = lax.dot_general(a, w3, (((1,), (1,)), ((), ())),
                        preferred_element_type=jnp.float32)
    hh = (g * jax.nn.sigmoid(g)) * u * jnp.float32(1.0 / NSH)
    part = lax.dot_general(hh, w2, (((1,), (1,)), ((), ())),
                           preferred_element_type=jnp.float32)

    @pl.when((s == 0) & (h == 0))
    def _():
        out_ref[...] = jnp.zeros_like(out_ref)

    out_ref[...] += part


def _shared_ffn(flat, sw1, sw2, sw3):
    grid = (N // MB, NSH, H // HB)
    return pl.pallas_call(
        _shared_body,
        grid=grid,
        in_specs=[
            pl.BlockSpec((MB, D), lambda m, s, h: (m, 0)),
            pl.BlockSpec((1, HB, D), lambda m, s, h: (s, h, 0)),
            pl.BlockSpec((1, HB, D), lambda m, s, h: (s, h, 0)),
            pl.BlockSpec((1, D, HB), lambda m, s, h: (s, 0, h)),
        ],
        out_specs=pl.BlockSpec((MB, D), lambda m, s, h: (m, 0)),
        out_shape=jax.ShapeDtypeStruct((N, D), jnp.float32),
        compiler_params=pltpu.CompilerParams(
            dimension_semantics=("arbitrary", "arbitrary", "arbitrary"),
        ),
    )(flat, sw1, sw3, sw2)


CCH = 16
CPW = N // NW


def _combine_body(yw_hbm, base_hbm, d0_hbm, d1_hbm, out_hbm,
                  d0_v, d1_v, y0_v, y1_v, acc_v, sem):
    wid = lax.axis_index("s") * 2 + lax.axis_index("c")
    base = wid * CPW

    def body(c, carry):
        off = base + c * CCH
        pltpu.sync_copy(d0_hbm.at[pl.ds(off, CCH)], d0_v)
        pltpu.sync_copy(d1_hbm.at[pl.ds(off, CCH)], d1_v)
        pltpu.async_copy(yw_hbm.at[d0_v], y0_v, sem).wait()
        pltpu.async_copy(yw_hbm.at[d1_v], y1_v, sem).wait()
        pltpu.sync_copy(base_hbm.at[pl.ds(off, CCH)], acc_v)

        def add_row(r, carry2):
            for gbase in range(0, D // 16, 1):
                sl = pl.ds(gbase * 16, 16)
                acc_v[r, sl] = acc_v[r, sl] + y0_v[r, sl] + y1_v[r, sl]
            return carry2

        lax.fori_loop(0, CCH, add_row, 0)
        pltpu.sync_copy(acc_v, out_hbm.at[pl.ds(off, CCH)])
        return carry

    lax.fori_loop(0, CPW // CCH, body, 0)


def _combine(yw, basev, d0, d1):
    f = functools.partial(
        pl.kernel,
        out_type=jax.ShapeDtypeStruct((N, D), jnp.float32),
        mesh=plsc.VectorSubcoreMesh(core_axis_name="c", subcore_axis_name="s"),
        scratch_types=[
            pltpu.VMEM((CCH,), jnp.int32),
            pltpu.VMEM((CCH,), jnp.int32),
            pltpu.VMEM((CCH, D), jnp.float32),
            pltpu.VMEM((CCH, D), jnp.float32),
            pltpu.VMEM((CCH, D), jnp.float32),
            pltpu.SemaphoreType.DMA,
        ],
    )(_combine_body)
    return f(yw, basev, d0, d1)


@jax.jit
def kernel(x, gate_w, gate_b, ew1, ew2, ew3, sw1, sw2, sw3):
    Bb, Tt, Dd = x.shape
    flat = x.reshape(-1, Dd)
    d0, d1, idxl3, wslot = _router(flat, gate_w)
    xg = _dispatch(flat, idxl3.reshape(TOT))
    basev = _shared_ffn(flat, sw1, sw2, sw3)
    yw = _expert_ffn(xg, ew1, ew2, ew3, wslot)
    out = _combine(yw, basev, d0, d1)
    return out.reshape(Bb, Tt, Dd)
